# baseline (speedup 1.0000x reference)
import functools

import jax
import jax.numpy as jnp
from jax import lax
from jax.experimental import pallas as pl
from jax.experimental.pallas import tpu as pltpu
from jax.experimental.pallas import tpu_sc as plsc

_NBUF = 12


def _gather_kernel(n_total, d_model, b_per_w, chunk, n_chunks):
    mesh = plsc.VectorSubcoreMesh(core_axis_name="c", subcore_axis_name="s")

    @functools.partial(
        pl.kernel,
        mesh=mesh,
        out_type=jax.ShapeDtypeStruct((n_total, d_model), jnp.float32),
        scratch_types=[
            pltpu.VMEM((n_chunks, chunk), jnp.int32),
            pltpu.VMEM((_NBUF, chunk, d_model), jnp.float32),
            pltpu.SemaphoreType.DMA((_NBUF,)),
            pltpu.SemaphoreType.DMA((_NBUF,)),
        ],
    )
    def k(table_hbm, idx_hbm, out_hbm, idx_v, rows_v, gsem, ssem):
        nc = plsc.get_sparse_core_info().num_cores
        wid = lax.axis_index("s") * nc + lax.axis_index("c")
        base = wid * b_per_w
        pltpu.sync_copy(idx_hbm.at[wid], idx_v)

        def gather(c, b):
            cp = pltpu.make_async_copy(
                table_hbm.at[idx_v.at[c]], rows_v.at[b], gsem.at[b]
            )
            cp.start()
            return cp

        def scatter(c, b):
            pltpu.make_async_copy(
                table_hbm.at[idx_v.at[c]], rows_v.at[b], gsem.at[b]
            ).wait()
            cp = pltpu.make_async_copy(
                rows_v.at[b],
                out_hbm.at[pl.ds(base + c * chunk, chunk)],
                ssem.at[b],
            )
            cp.start()
            return cp

        def wait_scatter(c, b):
            pltpu.make_async_copy(
                rows_v.at[b],
                out_hbm.at[pl.ds(base + c * chunk, chunk)],
                ssem.at[b],
            ).wait()

        for c in range(_NBUF):
            gather(c, c)
        scatter(0, 0)

        n_groups = (n_chunks - _NBUF) // _NBUF
        c_end = n_groups * _NBUF

        def body(j):
            c0 = 1 + j * _NBUF
            for u in range(_NBUF):
                c = c0 + u
                scatter(c, (1 + u) % _NBUF)
                wait_scatter(c - 1, u % _NBUF)
                gather(c + _NBUF - 1, u % _NBUF)

        pl.loop(0, n_groups)(body)

        for c in range(c_end + 1, n_chunks):
            scatter(c, c % _NBUF)
            wait_scatter(c - 1, (c - 1) % _NBUF)
            if c + _NBUF - 1 < n_chunks:
                gather(c + _NBUF - 1, (c - 1) % _NBUF)
        wait_scatter(n_chunks - 1, (n_chunks - 1) % _NBUF)

    return k


def kernel(x, pe):
    b, s = x.shape
    v, d = pe.shape
    n = b * s
    info = plsc.get_sparse_core_info()
    nw = info.num_cores * info.num_subcores
    b_per_w = n // nw
    chunk = 8
    n_chunks = b_per_w // chunk
    idx3 = x.astype(jnp.int32).reshape(nw, n_chunks, chunk)
    k = _gather_kernel(n, d, b_per_w, chunk, n_chunks)
    out = k(pe, idx3)
    return out.reshape(b, s, d)


# speedup vs baseline: 1.0055x; 1.0055x over previous
import functools

import jax
import jax.numpy as jnp
from jax import lax
from jax.experimental import pallas as pl
from jax.experimental.pallas import tpu as pltpu
from jax.experimental.pallas import tpu_sc as plsc

_NBUF = 6


def _gather_kernel(n_total, d_model, b_per_w, chunk, n_chunks):
    mesh = plsc.VectorSubcoreMesh(core_axis_name="c", subcore_axis_name="s")

    @functools.partial(
        pl.kernel,
        mesh=mesh,
        out_type=jax.ShapeDtypeStruct((n_total, d_model), jnp.float32),
        scratch_types=[
            pltpu.VMEM((n_chunks, chunk), jnp.int32),
            pltpu.VMEM((_NBUF, chunk, d_model), jnp.float32),
            pltpu.SemaphoreType.DMA((_NBUF,)),
            pltpu.SemaphoreType.DMA((_NBUF,)),
        ],
    )
    def k(table_hbm, idx_hbm, out_hbm, idx_v, rows_v, gsem, ssem):
        nc = plsc.get_sparse_core_info().num_cores
        nw = nc * plsc.get_sparse_core_info().num_subcores
        wid = lax.axis_index("s") * nc + lax.axis_index("c")
        pltpu.sync_copy(idx_hbm.at[wid], idx_v)

        def out_base(c):
            return (c * nw + wid) * chunk

        def gather(c, b):
            cp = pltpu.make_async_copy(
                table_hbm.at[idx_v.at[c]], rows_v.at[b], gsem.at[b]
            )
            cp.start()
            return cp

        def scatter(c, b):
            pltpu.make_async_copy(
                table_hbm.at[idx_v.at[c]], rows_v.at[b], gsem.at[b]
            ).wait()
            cp = pltpu.make_async_copy(
                rows_v.at[b],
                out_hbm.at[pl.ds(out_base(c), chunk)],
                ssem.at[b],
            )
            cp.start()
            return cp

        def wait_scatter(c, b):
            pltpu.make_async_copy(
                rows_v.at[b],
                out_hbm.at[pl.ds(out_base(c), chunk)],
                ssem.at[b],
            ).wait()

        for c in range(_NBUF):
            gather(c, c)
        scatter(0, 0)

        n_groups = (n_chunks - _NBUF) // _NBUF
        c_end = n_groups * _NBUF

        def body(j):
            c0 = 1 + j * _NBUF
            for u in range(_NBUF):
                c = c0 + u
                scatter(c, (1 + u) % _NBUF)
                wait_scatter(c - 1, u % _NBUF)
                gather(c + _NBUF - 1, u % _NBUF)

        pl.loop(0, n_groups)(body)

        for c in range(c_end + 1, n_chunks):
            scatter(c, c % _NBUF)
            wait_scatter(c - 1, (c - 1) % _NBUF)
            if c + _NBUF - 1 < n_chunks:
                gather(c + _NBUF - 1, (c - 1) % _NBUF)
        wait_scatter(n_chunks - 1, (n_chunks - 1) % _NBUF)

    return k


def kernel(x, pe):
    b, s = x.shape
    v, d = pe.shape
    n = b * s
    info = plsc.get_sparse_core_info()
    nw = info.num_cores * info.num_subcores
    b_per_w = n // nw
    chunk = 16
    n_chunks = b_per_w // chunk
    idx3 = x.astype(jnp.int32).reshape(n_chunks, nw, chunk).transpose(1, 0, 2)
    k = _gather_kernel(n, d, b_per_w, chunk, n_chunks)
    out = k(pe, idx3)
    return out.reshape(b, s, d)


# speedup vs baseline: 1.0085x; 1.0030x over previous
import functools

import jax
import jax.numpy as jnp
from jax import lax
from jax.experimental import pallas as pl
from jax.experimental.pallas import tpu as pltpu
from jax.experimental.pallas import tpu_sc as plsc

_NBUF = 6


def _gather_kernel(n_total, d_model, b_per_w, chunk, n_chunks):
    mesh = plsc.VectorSubcoreMesh(core_axis_name="c", subcore_axis_name="s")

    @functools.partial(
        pl.kernel,
        mesh=mesh,
        out_type=jax.ShapeDtypeStruct((n_total, d_model), jnp.float32),
        scratch_types=[
            pltpu.VMEM((n_chunks, chunk), jnp.int32),
            pltpu.VMEM((_NBUF, chunk, d_model), jnp.float32),
            pltpu.SemaphoreType.DMA((_NBUF,)),
            pltpu.SemaphoreType.DMA((_NBUF,)),
        ],
    )
    def k(table_hbm, idx_hbm, out_hbm, idx_v, rows_v, gsem, ssem):
        nc = plsc.get_sparse_core_info().num_cores
        wid = lax.axis_index("s") * nc + lax.axis_index("c")
        base = wid * b_per_w
        pltpu.sync_copy(idx_hbm.at[wid], idx_v)

        def gather(c, b):
            cp = pltpu.make_async_copy(
                table_hbm.at[idx_v.at[c]], rows_v.at[b], gsem.at[b]
            )
            cp.start()
            return cp

        def scatter(c, b):
            pltpu.make_async_copy(
                table_hbm.at[idx_v.at[c]], rows_v.at[b], gsem.at[b]
            ).wait()
            cp = pltpu.make_async_copy(
                rows_v.at[b],
                out_hbm.at[pl.ds(base + c * chunk, chunk)],
                ssem.at[b],
            )
            cp.start()
            return cp

        def wait_scatter(c, b):
            pltpu.make_async_copy(
                rows_v.at[b],
                out_hbm.at[pl.ds(base + c * chunk, chunk)],
                ssem.at[b],
            ).wait()

        for c in range(_NBUF):
            gather(c, c)
        scatter(0, 0)

        n_groups = (n_chunks - _NBUF) // _NBUF
        c_end = n_groups * _NBUF

        def body(j):
            c0 = 1 + j * _NBUF
            for u in range(_NBUF):
                c = c0 + u
                scatter(c, (1 + u) % _NBUF)
                wait_scatter(c - 1, u % _NBUF)
                gather(c + _NBUF - 1, u % _NBUF)

        pl.loop(0, n_groups)(body)

        for c in range(c_end + 1, n_chunks):
            scatter(c, c % _NBUF)
            wait_scatter(c - 1, (c - 1) % _NBUF)
            if c + _NBUF - 1 < n_chunks:
                gather(c + _NBUF - 1, (c - 1) % _NBUF)
        wait_scatter(n_chunks - 1, (n_chunks - 1) % _NBUF)

    return k


def kernel(x, pe):
    b, s = x.shape
    v, d = pe.shape
    n = b * s
    info = plsc.get_sparse_core_info()
    nw = info.num_cores * info.num_subcores
    b_per_w = n // nw
    chunk = 16
    n_chunks = b_per_w // chunk
    idx3 = x.astype(jnp.int32).reshape(nw, n_chunks, chunk)
    k = _gather_kernel(n, d, b_per_w, chunk, n_chunks)
    out = k(pe, idx3)
    return out.reshape(b, s, d)
